# baseline (speedup 1.0000x reference)
import functools
import math

import jax
import jax.numpy as jnp
from jax import lax
from jax.experimental import pallas as pl
from jax.experimental.pallas import tpu as pltpu
from jax.experimental.pallas import tpu_sc as plsc

_NUM_SPECIES = 100
_CUTOFF = 5.0
_NUM_BASIS = 16
_N_NODES = 100000
_N_EDGES = 3200000
_EMBED_DIM = 8


_NC = 2
_NS = 16
_NW = _NC * _NS


def _sc_gather(table, idx_padded, b_per_w):
    n_pad = _NW * b_per_w
    d = table.shape[1]
    mesh = plsc.VectorSubcoreMesh(core_axis_name="c", subcore_axis_name="s")

    @functools.partial(
        pl.kernel,
        out_type=jax.ShapeDtypeStruct((n_pad, d), jnp.float32),
        mesh=mesh,
        scratch_types=[
            pltpu.VMEM((b_per_w,), jnp.int32),
            pltpu.VMEM((b_per_w, d), jnp.float32),
            pltpu.SemaphoreType.DMA,
        ],
        compiler_params=pltpu.CompilerParams(use_tc_tiling_on_sc=False),
    )
    def gather_kernel(table_hbm, idx_hbm, out_hbm, idx_v, rows_v, sem):
        wid = lax.axis_index("s") * _NC + lax.axis_index("c")
        base = wid * b_per_w
        pltpu.sync_copy(idx_hbm.at[pl.ds(base, b_per_w)], idx_v)
        pltpu.async_copy(table_hbm.at[idx_v], rows_v, sem).wait()
        pltpu.sync_copy(rows_v, out_hbm.at[pl.ds(base, b_per_w)])

    return gather_kernel(table, idx_padded)


def _bessel_kernel(ea_ref, out_ref, *, cutoff, num_basis):
    v = ea_ref[...]
    r2 = v[0] * v[0] + v[1] * v[1] + v[2] * v[2]
    r = jnp.sqrt(r2)
    coef = math.sqrt(2.0 / cutoff)
    y8 = r * (1.0 / cutoff)
    i8 = coef / r
    lane = lax.broadcasted_iota(jnp.int32, (8, 128), 1)
    sub = lax.broadcasted_iota(jnp.int32, (8, 128), 0)
    rep1 = (lane // num_basis == sub).astype(jnp.float32)
    rep_n = rep1 * (lane % num_basis + 1).astype(jnp.float32)
    y = jnp.dot(y8, rep_n, preferred_element_type=jnp.float32,
                precision=lax.Precision.HIGHEST)
    inv = jnp.dot(i8, rep1, preferred_element_type=jnp.float32,
                  precision=lax.Precision.HIGHEST)
    x = y - 2.0 * jnp.round(y * 0.5)
    x2 = x * x
    p = jnp.float32(-0.00614082)
    p = p * x2 + jnp.float32(0.08086612)
    p = p * x2 + jnp.float32(-0.59864496)
    p = p * x2 + jnp.float32(2.55002856)
    p = p * x2 + jnp.float32(-5.167702)
    p = p * x2 + jnp.float32(3.14159252)
    out_ref[...] = x * p * inv


def _edge_bessel(edge_attr, block_rows):
    e = edge_attr.shape[0]
    rows = e // 8
    grid = rows // block_rows
    ea3 = edge_attr.T.reshape(3, rows, 8)
    out = pl.pallas_call(
        functools.partial(_bessel_kernel, cutoff=_CUTOFF, num_basis=_NUM_BASIS),
        grid=(grid,),
        in_specs=[pl.BlockSpec((3, block_rows, 8), lambda i: (0, i, 0))],
        out_specs=pl.BlockSpec((block_rows, 128), lambda i: (i, 0)),
        out_shape=jax.ShapeDtypeStruct((rows, 128), jnp.float32),
    )(ea3)
    return out.reshape(e, _NUM_BASIS)


def kernel(x, edge_attr, W_node_x, W_node_z):
    b_per_w = 3128
    n_pad = _NW * b_per_w
    table = jnp.concatenate([W_node_x, W_node_z], axis=1)
    idx_padded = jnp.concatenate(
        [x, jnp.zeros((n_pad - _N_NODES,), jnp.int32)])
    h = _sc_gather(table, idx_padded, b_per_w)
    h_node_x = h[:_N_NODES, :_EMBED_DIM]
    h_node_z = h[:_N_NODES, _EMBED_DIM:]

    h_edge = _edge_bessel(edge_attr, block_rows=1600)

    return (h_node_x, h_node_z, h_edge)
